# baseline (speedup 1.0000x reference)
import functools

import jax
import jax.numpy as jnp
from jax import lax
from jax.experimental import pallas as pl
from jax.experimental.pallas import tpu as pltpu
from jax.experimental.pallas import tpu_sc as plsc

NC, NS = 2, 16
NW = NC * NS
C = 80
D = 128

NV = 10000
NE = 5000
NV_PAD = 10240
NE_PAD = 5120
NV_ACC = 10016
NE_ACC = 5008
NNZ = 320000
PW = NNZ // NW
NCH = PW // C
SLAB = PW


def _mesh():
  return plsc.VectorSubcoreMesh(
      core_axis_name="c", subcore_axis_name="s", num_cores=NC, num_subcores=NS)


def _zero_1d(ref, n):
  def body(i, _):
    ref[pl.ds(i * 16, 16)] = jnp.zeros((16,), jnp.float32)
    return 0
  lax.fori_loop(0, n // 16, body, 0)


def _zero_2d(ref, nrows, ncols):
  k = ncols // 16
  def body(i, _):
    r = i // k
    j = i % k
    ref[r, pl.ds(j * 16, 16)] = jnp.zeros((16,), jnp.float32)
    return 0
  lax.fori_loop(0, nrows * k, body, 0)


def _load_slab(src, slab, wid):
  pltpu.sync_copy(src.at[pl.ds(wid * PW, PW)], slab)


def _fill(dst, slab, j):
  def fb(k, _):
    dst[pl.ds(k * 16, 16)] = slab[pl.ds(j * C + k * 16, 16)]
    return 0
  lax.fori_loop(0, C // 16, fb, 0)


@functools.partial(
    pl.kernel,
    out_type=(jax.ShapeDtypeStruct((NC * NV_PAD,), jnp.float32),
              jax.ShapeDtypeStruct((NC * NE_PAD,), jnp.float32)),
    mesh=_mesh(),
    scratch_types=[
        pltpu.VMEM((SLAB,), jnp.int32),
        pltpu.VMEM((SLAB,), jnp.int32),
        pltpu.VMEM((C,), jnp.int32),
        pltpu.VMEM((C,), jnp.int32),
        pltpu.VMEM((C,), jnp.int32),
        pltpu.VMEM((C,), jnp.int32),
        pltpu.VMEM((C,), jnp.float32),
        pltpu.VMEM((NV_PAD // NS,), jnp.float32),
        pltpu.VMEM_SHARED((NV_PAD,), jnp.float32),
        pltpu.VMEM_SHARED((NE_PAD,), jnp.float32),
        pltpu.SemaphoreType.DMA,
        pltpu.SemaphoreType.DMA,
        pltpu.SemaphoreType.DMA,
        pltpu.SemaphoreType.DMA,
    ],
)
def _hist(nidx, hidx, dv_out, de_out,
          nl, hl, in0, in1, ih0, ih1, ones_v, zeros_v, dv_s, de_s,
          sn0, sn1, sh0, sh1):
  c = lax.axis_index("c")
  s = lax.axis_index("s")
  wid = s * NC + c

  _load_slab(nidx, nl, wid)
  _load_slab(hidx, hl, wid)
  _zero_1d(zeros_v, NV_PAD // NS)

  def ob(i, _):
    ones_v[pl.ds(i * 16, 16)] = jnp.ones((16,), jnp.float32)
    return 0
  lax.fori_loop(0, C // 16, ob, 0)

  vslice = NV_PAD // NS
  eslice = NE_PAD // NS
  pltpu.sync_copy(zeros_v, dv_s.at[pl.ds(s * vslice, vslice)])
  pltpu.sync_copy(zeros_v.at[pl.ds(0, eslice)], de_s.at[pl.ds(s * eslice, eslice)])
  plsc.subcore_barrier()

  def slot_step(j, me_in, me_ih, me_sn, me_sh):
    @pl.when(j >= 2)
    def _():
      pltpu.make_async_copy(ones_v, dv_s.at[me_in], me_sn).wait()
      pltpu.make_async_copy(ones_v, de_s.at[me_ih], me_sh).wait()
    _fill(me_in, nl, j)
    _fill(me_ih, hl, j)
    pltpu.async_copy(ones_v, dv_s.at[me_in], me_sn, add=True)
    pltpu.async_copy(ones_v, de_s.at[me_ih], me_sh, add=True)

  def body(j, _):
    @pl.when(j % 2 == 0)
    def _():
      slot_step(j, in0, ih0, sn0, sh0)
    @pl.when(j % 2 == 1)
    def _():
      slot_step(j, in1, ih1, sn1, sh1)
    return 0
  lax.fori_loop(0, NCH, body, 0)

  pltpu.make_async_copy(ones_v, dv_s.at[in0], sn0).wait()
  pltpu.make_async_copy(ones_v, de_s.at[ih0], sh0).wait()
  pltpu.make_async_copy(ones_v, dv_s.at[in1], sn1).wait()
  pltpu.make_async_copy(ones_v, de_s.at[ih1], sh1).wait()
  plsc.subcore_barrier()

  pltpu.sync_copy(dv_s.at[pl.ds(s * vslice, vslice)], zeros_v)
  pltpu.sync_copy(zeros_v, dv_out.at[pl.ds(c * NV_PAD + s * vslice, vslice)])
  pltpu.sync_copy(de_s.at[pl.ds(s * eslice, eslice)],
                  zeros_v.at[pl.ds(0, eslice)])
  pltpu.sync_copy(zeros_v.at[pl.ds(0, eslice)],
                  de_out.at[pl.ds(c * NE_PAD + s * eslice, eslice)])


def _proj_body(x_ref, w_ref, b_ref, dvp_ref, dep_ref, hs_ref, dvi_ref, dei_ref):
  dv = dvp_ref[0] + dvp_ref[1]
  dvi = jnp.where(dv > 0, lax.rsqrt(dv), 0.0)
  de = dep_ref[0] + dep_ref[1]
  dei = jnp.where(de > 0, 1.0 / de, 0.0)
  h = lax.dot_general(x_ref[...], w_ref[...], (((1,), (1,)), ((), ())),
                      preferred_element_type=jnp.float32,
                      precision=lax.Precision.HIGHEST)
  hs_ref[...] = (h + b_ref[...]) * dvi
  dvi_ref[...] = dvi
  dei_ref[...] = dei


def _proj(x, w, b2, dvp, dep):
  return pl.pallas_call(
      _proj_body,
      out_shape=(jax.ShapeDtypeStruct((NV, D), jnp.float32),
                 jax.ShapeDtypeStruct((NV, 1), jnp.float32),
                 jax.ShapeDtypeStruct((NE, 1), jnp.float32)),
  )(x, w, b2, dvp, dep)


def _make_agg(acc_rows, nslots):
  @functools.partial(
      pl.kernel,
      out_type=jax.ShapeDtypeStruct((NC, acc_rows, D), jnp.float32),
      mesh=_mesh(),
      scratch_types=(
          [pltpu.VMEM((SLAB,), jnp.int32)] * 2
          + [pltpu.VMEM((C,), jnp.int32)] * nslots
          + [pltpu.VMEM((C, D), jnp.float32)] * nslots
          + [pltpu.VMEM_SHARED((acc_rows, D), jnp.float32)]
          + [pltpu.SemaphoreType.DMA] * (2 * nslots)
      ),
  )
  def agg(table, gidx, sidx, out, gl, sl, *bufs):
    sbufs = bufs[0:nslots]
    rowss = bufs[nslots:2 * nslots]
    acc_s = bufs[2 * nslots]
    gsems = bufs[2 * nslots + 1:3 * nslots + 1]
    ssems = bufs[3 * nslots + 1:4 * nslots + 1]

    def gidx_of(j):
      return gl.at[pl.ds(j * C, C)]
    c = lax.axis_index("c")
    s = lax.axis_index("s")
    wid = s * NC + c

    _load_slab(gidx, gl, wid)
    _load_slab(sidx, sl, wid)
    _fill(sbufs[0], sl, 0)
    pltpu.async_copy(table.at[gidx_of(0)], rowss[0], gsems[0])

    _zero_2d(rowss[1], C, D)
    bpt = 8 * (-(-acc_rows // (8 * NS)))
    last = acc_rows - (NS - 1) * bpt

    def _span(base, nrows, fn):
      full, tail = nrows // C, nrows % C
      def sb(i, _):
        fn(base + i * C, C)
        return 0
      lax.fori_loop(0, full, sb, 0)
      if tail:
        fn(base + full * C, tail)

    def _zero_at(off, n):
      pltpu.sync_copy(rowss[1].at[pl.ds(0, n)], acc_s.at[pl.ds(off, n)])

    @pl.when(s < NS - 1)
    def _():
      _span(s * bpt, bpt, _zero_at)
    @pl.when(s == NS - 1)
    def _():
      _span((NS - 1) * bpt, last, _zero_at)
    plsc.subcore_barrier()

    slots = tuple((sbufs[p], rowss[p], gsems[p], ssems[p])
                  for p in range(nslots))

    def slot_step(j, me, nx):
      me_s, me_rows, me_gs, me_ss = me
      nx_s, nx_rows, nx_gs, nx_ss = nx
      pltpu.make_async_copy(table.at[gidx_of(j)], me_rows, me_gs).wait()
      pltpu.async_copy(me_rows, acc_s.at[me_s], me_ss, add=True)
      @pl.when(j + 1 < NCH)
      def _():
        @pl.when(j >= nslots - 1)
        def _():
          pltpu.make_async_copy(nx_rows, acc_s.at[nx_s], nx_ss).wait()
        _fill(nx_s, sl, j + 1)
        pltpu.async_copy(table.at[gidx_of(j + 1)], nx_rows, nx_gs)

    def body(j, _):
      for p in range(nslots):
        @pl.when(j % nslots == p)
        def _(p=p):
          slot_step(j, slots[p], slots[(p + 1) % nslots])
      return 0
    lax.fori_loop(0, NCH, body, 0)

    for p in range(nslots):
      s_p, rows_p, gs_p, ss_p = slots[p]
      pltpu.make_async_copy(rows_p, acc_s.at[s_p], ss_p).wait()
    plsc.subcore_barrier()

    def _dump_at(off, n):
      pltpu.sync_copy(acc_s.at[pl.ds(off, n)], rowss[0].at[pl.ds(0, n)])
      pltpu.sync_copy(rowss[0].at[pl.ds(0, n)], out.at[c, pl.ds(off, n)])

    @pl.when(s < NS - 1)
    def _():
      _span(s * bpt, bpt, _dump_at)
    @pl.when(s == NS - 1)
    def _():
      _span((NS - 1) * bpt, last, _dump_at)

  return agg


_hedge_agg = _make_agg(NE_ACC, 2)
_node_agg = _make_agg(NV_ACC, 2)


def _ye_combine_body(yep_ref, dei_ref, ye_ref):
  ye = yep_ref[0, :NE, :] + yep_ref[1, :NE, :]
  ye_ref[...] = ye * dei_ref[...]


def _ye_combine(yep, dei):
  return pl.pallas_call(
      _ye_combine_body,
      out_shape=jax.ShapeDtypeStruct((NE, D), jnp.float32),
  )(yep, dei)


def _z_combine_body(zp_ref, dvi_ref, z_ref):
  z = zp_ref[0, :NV, :] + zp_ref[1, :NV, :]
  z_ref[...] = jnp.maximum(z * dvi_ref[...], 0.0)


def _z_combine(zp, dvi):
  return pl.pallas_call(
      _z_combine_body,
      out_shape=jax.ShapeDtypeStruct((NV, D), jnp.float32),
  )(zp, dvi)


def kernel(X, node_idx, hedge_idx, W, b):
  nidx = node_idx.astype(jnp.int32)
  hidx = hedge_idx.astype(jnp.int32)
  dvp, dep = _hist(nidx, hidx)
  dvp = dvp.reshape(NC, NV_PAD)[:, :NV, None]
  dep = dep.reshape(NC, NE_PAD)[:, :NE, None]
  hs, dvi, dei = _proj(X, W, b.reshape(1, D), dvp, dep)
  yep = _hedge_agg(hs, nidx, hidx)
  ye = _ye_combine(yep, dei)
  zp = _node_agg(ye, hidx, nidx)
  return _z_combine(zp, dvi)


# speedup vs baseline: 1.2543x; 1.2543x over previous
import functools

import jax
import jax.numpy as jnp
from jax import lax
from jax.experimental import pallas as pl
from jax.experimental.pallas import tpu as pltpu
from jax.experimental.pallas import tpu_sc as plsc

NC, NS = 2, 16
NW = NC * NS
C = 80
D = 128

NV = 10000
NE = 5000
NV_PAD = 10240
NE_PAD = 5120
NV_ACC = 10016
NE_ACC = 5008
NNZ = 320000
PW = NNZ // NW
NCH = PW // C
SLAB = PW


def _mesh():
  return plsc.VectorSubcoreMesh(
      core_axis_name="c", subcore_axis_name="s", num_cores=NC, num_subcores=NS)


def _zero_1d(ref, n):
  def body(i, _):
    ref[pl.ds(i * 16, 16)] = jnp.zeros((16,), jnp.float32)
    return 0
  lax.fori_loop(0, n // 16, body, 0)


def _zero_2d(ref, nrows, ncols):
  k = ncols // 16
  def body(i, _):
    r = i // k
    j = i % k
    ref[r, pl.ds(j * 16, 16)] = jnp.zeros((16,), jnp.float32)
    return 0
  lax.fori_loop(0, nrows * k, body, 0)


def _load_slab(src, slab, wid):
  pltpu.sync_copy(src.at[pl.ds(wid * PW, PW)], slab)


def _fill(dst, slab, j):
  def fb(k, _):
    dst[pl.ds(k * 16, 16)] = slab[pl.ds(j * C + k * 16, 16)]
    return 0
  lax.fori_loop(0, C // 16, fb, 0)


@functools.partial(
    pl.kernel,
    out_type=(jax.ShapeDtypeStruct((NC * NV_PAD,), jnp.float32),
              jax.ShapeDtypeStruct((NC * NE_PAD,), jnp.float32)),
    mesh=_mesh(),
    scratch_types=[
        pltpu.VMEM((SLAB,), jnp.int32),
        pltpu.VMEM((SLAB,), jnp.int32),
        pltpu.VMEM((C,), jnp.int32),
        pltpu.VMEM((C,), jnp.int32),
        pltpu.VMEM((C,), jnp.int32),
        pltpu.VMEM((C,), jnp.int32),
        pltpu.VMEM((C,), jnp.float32),
        pltpu.VMEM((NV_PAD // NS,), jnp.float32),
        pltpu.VMEM_SHARED((NV_PAD,), jnp.float32),
        pltpu.VMEM_SHARED((NE_PAD,), jnp.float32),
        pltpu.SemaphoreType.DMA,
        pltpu.SemaphoreType.DMA,
        pltpu.SemaphoreType.DMA,
        pltpu.SemaphoreType.DMA,
    ],
)
def _hist(nidx, hidx, dv_out, de_out,
          nl, hl, in0, in1, ih0, ih1, ones_v, zeros_v, dv_s, de_s,
          sn0, sn1, sh0, sh1):
  c = lax.axis_index("c")
  s = lax.axis_index("s")
  wid = s * NC + c

  _load_slab(nidx, nl, wid)
  _load_slab(hidx, hl, wid)
  _zero_1d(zeros_v, NV_PAD // NS)

  def ob(i, _):
    ones_v[pl.ds(i * 16, 16)] = jnp.ones((16,), jnp.float32)
    return 0
  lax.fori_loop(0, C // 16, ob, 0)

  vslice = NV_PAD // NS
  eslice = NE_PAD // NS
  pltpu.sync_copy(zeros_v, dv_s.at[pl.ds(s * vslice, vslice)])
  pltpu.sync_copy(zeros_v.at[pl.ds(0, eslice)], de_s.at[pl.ds(s * eslice, eslice)])
  plsc.subcore_barrier()

  def slot_step(j, me_in, me_ih, me_sn, me_sh):
    @pl.when(j >= 2)
    def _():
      pltpu.make_async_copy(ones_v, dv_s.at[me_in], me_sn).wait()
      pltpu.make_async_copy(ones_v, de_s.at[me_ih], me_sh).wait()
    _fill(me_in, nl, j)
    _fill(me_ih, hl, j)
    pltpu.async_copy(ones_v, dv_s.at[me_in], me_sn, add=True)
    pltpu.async_copy(ones_v, de_s.at[me_ih], me_sh, add=True)

  def body(j, _):
    @pl.when(j % 2 == 0)
    def _():
      slot_step(j, in0, ih0, sn0, sh0)
    @pl.when(j % 2 == 1)
    def _():
      slot_step(j, in1, ih1, sn1, sh1)
    return 0
  lax.fori_loop(0, NCH, body, 0)

  pltpu.make_async_copy(ones_v, dv_s.at[in0], sn0).wait()
  pltpu.make_async_copy(ones_v, de_s.at[ih0], sh0).wait()
  pltpu.make_async_copy(ones_v, dv_s.at[in1], sn1).wait()
  pltpu.make_async_copy(ones_v, de_s.at[ih1], sh1).wait()
  plsc.subcore_barrier()

  pltpu.sync_copy(dv_s.at[pl.ds(s * vslice, vslice)], zeros_v)
  pltpu.sync_copy(zeros_v, dv_out.at[pl.ds(c * NV_PAD + s * vslice, vslice)])
  pltpu.sync_copy(de_s.at[pl.ds(s * eslice, eslice)],
                  zeros_v.at[pl.ds(0, eslice)])
  pltpu.sync_copy(zeros_v.at[pl.ds(0, eslice)],
                  de_out.at[pl.ds(c * NE_PAD + s * eslice, eslice)])


def _proj_body(x_ref, w_ref, b_ref, dvp_ref, dep_ref, hs_ref, dvi_ref, dei_ref):
  dv = dvp_ref[0] + dvp_ref[1]
  dvi = jnp.where(dv > 0, lax.rsqrt(dv), 0.0)
  de = dep_ref[0] + dep_ref[1]
  dei = jnp.where(de > 0, 1.0 / de, 0.0)
  h = lax.dot_general(x_ref[...], w_ref[...], (((1,), (1,)), ((), ())),
                      preferred_element_type=jnp.float32,
                      precision=lax.Precision.HIGHEST)
  hs_ref[...] = (h + b_ref[...]) * dvi
  dvi_ref[...] = dvi
  dei_ref[...] = dei


def _proj(x, w, b2, dvp, dep):
  return pl.pallas_call(
      _proj_body,
      out_shape=(jax.ShapeDtypeStruct((NV, D), jnp.float32),
                 jax.ShapeDtypeStruct((NV, 1), jnp.float32),
                 jax.ShapeDtypeStruct((NE, 1), jnp.float32)),
  )(x, w, b2, dvp, dep)


def _make_agg(acc_rows, nslots):
  @functools.partial(
      pl.kernel,
      out_type=jax.ShapeDtypeStruct((NC, acc_rows, D), jnp.float32),
      mesh=_mesh(),
      scratch_types=(
          [pltpu.VMEM((SLAB,), jnp.int32)] * 2
          + [pltpu.VMEM((C,), jnp.int32)] * (2 * nslots)
          + [pltpu.VMEM((C, D), jnp.float32)] * nslots
          + [pltpu.VMEM_SHARED((acc_rows, D), jnp.float32)]
          + [pltpu.SemaphoreType.DMA] * (2 * nslots)
      ),
  )
  def agg(table, gidx, sidx, out, gl, sl, *bufs):
    gbufs = bufs[0:nslots]
    sbufs = bufs[nslots:2 * nslots]
    rowss = bufs[2 * nslots:3 * nslots]
    acc_s = bufs[3 * nslots]
    gsems = bufs[3 * nslots + 1:4 * nslots + 1]
    ssems = bufs[4 * nslots + 1:5 * nslots + 1]
    c = lax.axis_index("c")
    s = lax.axis_index("s")
    wid = s * NC + c

    _load_slab(gidx, gl, wid)
    _load_slab(sidx, sl, wid)
    _fill(gbufs[0], gl, 0)
    _fill(sbufs[0], sl, 0)
    pltpu.async_copy(table.at[gbufs[0]], rowss[0], gsems[0])

    _zero_2d(rowss[1], C, D)
    bpt = 8 * (-(-acc_rows // (8 * NS)))
    last = acc_rows - (NS - 1) * bpt

    def _span(base, nrows, fn):
      full, tail = nrows // C, nrows % C
      def sb(i, _):
        fn(base + i * C, C)
        return 0
      lax.fori_loop(0, full, sb, 0)
      if tail:
        fn(base + full * C, tail)

    def _zero_at(off, n):
      pltpu.sync_copy(rowss[1].at[pl.ds(0, n)], acc_s.at[pl.ds(off, n)])

    @pl.when(s < NS - 1)
    def _():
      _span(s * bpt, bpt, _zero_at)
    @pl.when(s == NS - 1)
    def _():
      _span((NS - 1) * bpt, last, _zero_at)
    plsc.subcore_barrier()

    slots = tuple((gbufs[p], sbufs[p], rowss[p], gsems[p], ssems[p])
                  for p in range(nslots))

    def slot_step(j, me, nx):
      me_g, me_s, me_rows, me_gs, me_ss = me
      nx_g, nx_s, nx_rows, nx_gs, nx_ss = nx
      @pl.when(j + 1 < NCH)
      def _():
        @pl.when(j >= nslots - 1)
        def _():
          pltpu.make_async_copy(nx_rows, acc_s.at[nx_s], nx_ss).wait()
        _fill(nx_g, gl, j + 1)
        _fill(nx_s, sl, j + 1)
        pltpu.async_copy(table.at[nx_g], nx_rows, nx_gs)
      pltpu.make_async_copy(table.at[me_g], me_rows, me_gs).wait()
      pltpu.async_copy(me_rows, acc_s.at[me_s], me_ss, add=True)

    def body(j, _):
      for p in range(nslots):
        @pl.when(j % nslots == p)
        def _(p=p):
          slot_step(j, slots[p], slots[(p + 1) % nslots])
      return 0
    lax.fori_loop(0, NCH, body, 0)

    for p in range(nslots):
      g_p, s_p, rows_p, gs_p, ss_p = slots[p]
      pltpu.make_async_copy(rows_p, acc_s.at[s_p], ss_p).wait()
    plsc.subcore_barrier()

    def _dump_at(off, n):
      pltpu.sync_copy(acc_s.at[pl.ds(off, n)], rowss[0].at[pl.ds(0, n)])
      pltpu.sync_copy(rowss[0].at[pl.ds(0, n)], out.at[c, pl.ds(off, n)])

    @pl.when(s < NS - 1)
    def _():
      _span(s * bpt, bpt, _dump_at)
    @pl.when(s == NS - 1)
    def _():
      _span((NS - 1) * bpt, last, _dump_at)

  return agg


_hedge_agg = _make_agg(NE_ACC, 2)
_node_agg = _make_agg(NV_ACC, 2)


def _ye_combine_body(yep_ref, dei_ref, ye_ref):
  ye = yep_ref[0, :NE, :] + yep_ref[1, :NE, :]
  ye_ref[...] = ye * dei_ref[...]


def _ye_combine(yep, dei):
  return pl.pallas_call(
      _ye_combine_body,
      out_shape=jax.ShapeDtypeStruct((NE, D), jnp.float32),
  )(yep, dei)


def _z_combine_body(zp_ref, dvi_ref, z_ref):
  z = zp_ref[0, :NV, :] + zp_ref[1, :NV, :]
  z_ref[...] = jnp.maximum(z * dvi_ref[...], 0.0)


def _z_combine(zp, dvi):
  return pl.pallas_call(
      _z_combine_body,
      out_shape=jax.ShapeDtypeStruct((NV, D), jnp.float32),
  )(zp, dvi)


def kernel(X, node_idx, hedge_idx, W, b):
  nidx = node_idx.astype(jnp.int32)
  hidx = hedge_idx.astype(jnp.int32)
  dvp, dep = _hist(nidx, hidx)
  dvp = dvp.reshape(NC, NV_PAD)[:, :NV, None]
  dep = dep.reshape(NC, NE_PAD)[:, :NE, None]
  hs, dvi, dei = _proj(X, W, b.reshape(1, D), dvp, dep)
  yep = _hedge_agg(hs, nidx, hidx)
  ye = _ye_combine(yep, dei)
  zp = _node_agg(ye, hidx, nidx)
  return _z_combine(zp, dvi)


# speedup vs baseline: 1.2569x; 1.0020x over previous
import functools

import jax
import jax.numpy as jnp
from jax import lax
from jax.experimental import pallas as pl
from jax.experimental.pallas import tpu as pltpu
from jax.experimental.pallas import tpu_sc as plsc

NC, NS = 2, 16
NW = NC * NS
C = 80
D = 128

NV = 10000
NE = 5000
NV_PAD = 10240
NE_PAD = 5120
NV_ACC = 10016
NE_ACC = 5008
NNZ = 320000
PW = NNZ // NW
NCH = PW // C
SLAB = PW


def _mesh():
  return plsc.VectorSubcoreMesh(
      core_axis_name="c", subcore_axis_name="s", num_cores=NC, num_subcores=NS)


def _zero_1d(ref, n):
  def body(i, _):
    ref[pl.ds(i * 16, 16)] = jnp.zeros((16,), jnp.float32)
    return 0
  lax.fori_loop(0, n // 16, body, 0)


def _zero_2d(ref, nrows, ncols):
  k = ncols // 16
  def body(i, _):
    r = i // k
    j = i % k
    ref[r, pl.ds(j * 16, 16)] = jnp.zeros((16,), jnp.float32)
    return 0
  lax.fori_loop(0, nrows * k, body, 0)


def _load_slab(src, slab, wid):
  pltpu.sync_copy(src.at[pl.ds(wid * PW, PW)], slab)


def _fill(dst, slab, j):
  def fb(k, _):
    dst[pl.ds(k * 16, 16)] = slab[pl.ds(j * C + k * 16, 16)]
    return 0
  lax.fori_loop(0, C // 16, fb, 0)


@functools.partial(
    pl.kernel,
    out_type=(jax.ShapeDtypeStruct((NC * NV_PAD,), jnp.float32),
              jax.ShapeDtypeStruct((NC * NE_PAD,), jnp.float32)),
    mesh=_mesh(),
    scratch_types=[
        pltpu.VMEM((SLAB,), jnp.int32),
        pltpu.VMEM((SLAB,), jnp.int32),
        pltpu.VMEM((C,), jnp.int32),
        pltpu.VMEM((C,), jnp.int32),
        pltpu.VMEM((C,), jnp.int32),
        pltpu.VMEM((C,), jnp.int32),
        pltpu.VMEM((C,), jnp.float32),
        pltpu.VMEM((NV_PAD // NS,), jnp.float32),
        pltpu.VMEM_SHARED((NV_PAD,), jnp.float32),
        pltpu.VMEM_SHARED((NE_PAD,), jnp.float32),
        pltpu.SemaphoreType.DMA,
        pltpu.SemaphoreType.DMA,
        pltpu.SemaphoreType.DMA,
        pltpu.SemaphoreType.DMA,
    ],
)
def _hist(nidx, hidx, dv_out, de_out,
          nl, hl, in0, in1, ih0, ih1, ones_v, zeros_v, dv_s, de_s,
          sn0, sn1, sh0, sh1):
  c = lax.axis_index("c")
  s = lax.axis_index("s")
  wid = s * NC + c

  _load_slab(nidx, nl, wid)
  _load_slab(hidx, hl, wid)
  _zero_1d(zeros_v, NV_PAD // NS)

  def ob(i, _):
    ones_v[pl.ds(i * 16, 16)] = jnp.ones((16,), jnp.float32)
    return 0
  lax.fori_loop(0, C // 16, ob, 0)

  vslice = NV_PAD // NS
  eslice = NE_PAD // NS
  pltpu.sync_copy(zeros_v, dv_s.at[pl.ds(s * vslice, vslice)])
  pltpu.sync_copy(zeros_v.at[pl.ds(0, eslice)], de_s.at[pl.ds(s * eslice, eslice)])
  plsc.subcore_barrier()

  def slot_step(j, me_in, me_ih, me_sn, me_sh):
    @pl.when(j >= 2)
    def _():
      pltpu.make_async_copy(ones_v, dv_s.at[me_in], me_sn).wait()
      pltpu.make_async_copy(ones_v, de_s.at[me_ih], me_sh).wait()
    _fill(me_in, nl, j)
    _fill(me_ih, hl, j)
    pltpu.async_copy(ones_v, dv_s.at[me_in], me_sn, add=True)
    pltpu.async_copy(ones_v, de_s.at[me_ih], me_sh, add=True)

  def body(j, _):
    @pl.when(j % 2 == 0)
    def _():
      slot_step(j, in0, ih0, sn0, sh0)
    @pl.when(j % 2 == 1)
    def _():
      slot_step(j, in1, ih1, sn1, sh1)
    return 0
  lax.fori_loop(0, NCH, body, 0)

  pltpu.make_async_copy(ones_v, dv_s.at[in0], sn0).wait()
  pltpu.make_async_copy(ones_v, de_s.at[ih0], sh0).wait()
  pltpu.make_async_copy(ones_v, dv_s.at[in1], sn1).wait()
  pltpu.make_async_copy(ones_v, de_s.at[ih1], sh1).wait()
  plsc.subcore_barrier()

  pltpu.sync_copy(dv_s.at[pl.ds(s * vslice, vslice)], zeros_v)
  pltpu.sync_copy(zeros_v, dv_out.at[pl.ds(c * NV_PAD + s * vslice, vslice)])
  pltpu.sync_copy(de_s.at[pl.ds(s * eslice, eslice)],
                  zeros_v.at[pl.ds(0, eslice)])
  pltpu.sync_copy(zeros_v.at[pl.ds(0, eslice)],
                  de_out.at[pl.ds(c * NE_PAD + s * eslice, eslice)])


def _matmul_body(x_ref, w_ref, b_ref, h_ref):
  h = lax.dot_general(x_ref[...], w_ref[...], (((1,), (1,)), ((), ())),
                      preferred_element_type=jnp.float32,
                      precision=lax.Precision.HIGHEST)
  h_ref[...] = h + b_ref[...]


def _matmul(x, w, b2):
  return pl.pallas_call(
      _matmul_body,
      out_shape=jax.ShapeDtypeStruct((NV, D), jnp.float32),
  )(x, w, b2)


def _scale_body(h_ref, dvp_ref, dep_ref, hs_ref, dvi_ref, dei_ref):
  dv = dvp_ref[0] + dvp_ref[1]
  dvi = jnp.where(dv > 0, lax.rsqrt(dv), 0.0)
  de = dep_ref[0] + dep_ref[1]
  dei = jnp.where(de > 0, 1.0 / de, 0.0)
  hs_ref[...] = h_ref[...] * dvi
  dvi_ref[...] = dvi
  dei_ref[...] = dei


def _scale(h, dvp, dep):
  return pl.pallas_call(
      _scale_body,
      out_shape=(jax.ShapeDtypeStruct((NV, D), jnp.float32),
                 jax.ShapeDtypeStruct((NV, 1), jnp.float32),
                 jax.ShapeDtypeStruct((NE, 1), jnp.float32)),
  )(h, dvp, dep)


def _make_agg(acc_rows, nslots):
  @functools.partial(
      pl.kernel,
      out_type=jax.ShapeDtypeStruct((NC, acc_rows, D), jnp.float32),
      mesh=_mesh(),
      scratch_types=(
          [pltpu.VMEM((SLAB,), jnp.int32)] * 2
          + [pltpu.VMEM((C,), jnp.int32)] * (2 * nslots)
          + [pltpu.VMEM((C, D), jnp.float32)] * nslots
          + [pltpu.VMEM_SHARED((acc_rows, D), jnp.float32)]
          + [pltpu.SemaphoreType.DMA] * (2 * nslots)
      ),
  )
  def agg(table, gidx, sidx, out, gl, sl, *bufs):
    gbufs = bufs[0:nslots]
    sbufs = bufs[nslots:2 * nslots]
    rowss = bufs[2 * nslots:3 * nslots]
    acc_s = bufs[3 * nslots]
    gsems = bufs[3 * nslots + 1:4 * nslots + 1]
    ssems = bufs[4 * nslots + 1:5 * nslots + 1]
    c = lax.axis_index("c")
    s = lax.axis_index("s")
    wid = s * NC + c

    _load_slab(gidx, gl, wid)
    _load_slab(sidx, sl, wid)
    _fill(gbufs[0], gl, 0)
    _fill(sbufs[0], sl, 0)
    pltpu.async_copy(table.at[gbufs[0]], rowss[0], gsems[0])

    _zero_2d(rowss[1], C, D)
    bpt = 8 * (-(-acc_rows // (8 * NS)))
    last = acc_rows - (NS - 1) * bpt

    def _span(base, nrows, fn):
      full, tail = nrows // C, nrows % C
      def sb(i, _):
        fn(base + i * C, C)
        return 0
      lax.fori_loop(0, full, sb, 0)
      if tail:
        fn(base + full * C, tail)

    def _zero_at(off, n):
      pltpu.sync_copy(rowss[1].at[pl.ds(0, n)], acc_s.at[pl.ds(off, n)])

    @pl.when(s < NS - 1)
    def _():
      _span(s * bpt, bpt, _zero_at)
    @pl.when(s == NS - 1)
    def _():
      _span((NS - 1) * bpt, last, _zero_at)
    plsc.subcore_barrier()

    slots = tuple((gbufs[p], sbufs[p], rowss[p], gsems[p], ssems[p])
                  for p in range(nslots))

    def slot_step(j, me, nx):
      me_g, me_s, me_rows, me_gs, me_ss = me
      nx_g, nx_s, nx_rows, nx_gs, nx_ss = nx
      @pl.when(j + 1 < NCH)
      def _():
        @pl.when(j >= nslots - 1)
        def _():
          pltpu.make_async_copy(nx_rows, acc_s.at[nx_s], nx_ss).wait()
        _fill(nx_g, gl, j + 1)
        _fill(nx_s, sl, j + 1)
        pltpu.async_copy(table.at[nx_g], nx_rows, nx_gs)
      pltpu.make_async_copy(table.at[me_g], me_rows, me_gs).wait()
      pltpu.async_copy(me_rows, acc_s.at[me_s], me_ss, add=True)

    def body(j, _):
      for p in range(nslots):
        @pl.when(j % nslots == p)
        def _(p=p):
          slot_step(j, slots[p], slots[(p + 1) % nslots])
      return 0
    lax.fori_loop(0, NCH, body, 0)

    for p in range(nslots):
      g_p, s_p, rows_p, gs_p, ss_p = slots[p]
      pltpu.make_async_copy(rows_p, acc_s.at[s_p], ss_p).wait()
    plsc.subcore_barrier()

    def _dump_at(off, n):
      pltpu.sync_copy(acc_s.at[pl.ds(off, n)], rowss[0].at[pl.ds(0, n)])
      pltpu.sync_copy(rowss[0].at[pl.ds(0, n)], out.at[c, pl.ds(off, n)])

    @pl.when(s < NS - 1)
    def _():
      _span(s * bpt, bpt, _dump_at)
    @pl.when(s == NS - 1)
    def _():
      _span((NS - 1) * bpt, last, _dump_at)

  return agg


_hedge_agg = _make_agg(NE_ACC, 2)
_node_agg = _make_agg(NV_ACC, 2)


def _ye_combine_body(yep_ref, dei_ref, ye_ref):
  ye = yep_ref[0, :NE, :] + yep_ref[1, :NE, :]
  ye_ref[...] = ye * dei_ref[...]


def _ye_combine(yep, dei):
  return pl.pallas_call(
      _ye_combine_body,
      out_shape=jax.ShapeDtypeStruct((NE, D), jnp.float32),
  )(yep, dei)


def _z_combine_body(zp_ref, dvi_ref, z_ref):
  z = zp_ref[0, :NV, :] + zp_ref[1, :NV, :]
  z_ref[...] = jnp.maximum(z * dvi_ref[...], 0.0)


def _z_combine(zp, dvi):
  return pl.pallas_call(
      _z_combine_body,
      out_shape=jax.ShapeDtypeStruct((NV, D), jnp.float32),
  )(zp, dvi)


def kernel(X, node_idx, hedge_idx, W, b):
  nidx = node_idx.astype(jnp.int32)
  hidx = hedge_idx.astype(jnp.int32)
  h = _matmul(X, W, b.reshape(1, D))
  dvp, dep = _hist(nidx, hidx)
  dvp = dvp.reshape(NC, NV_PAD)[:, :NV, None]
  dep = dep.reshape(NC, NE_PAD)[:, :NE, None]
  hs, dvi, dei = _scale(h, dvp, dep)
  yep = _hedge_agg(hs, nidx, hidx)
  ye = _ye_combine(yep, dei)
  zp = _node_agg(ye, hidx, nidx)
  return _z_combine(zp, dvi)
